# baseline (speedup 1.0000x reference)
import functools
import math

import jax
import jax.numpy as jnp
from jax import lax
from jax.experimental import pallas as pl
from jax.experimental.pallas import tpu as pltpu
from jax.experimental.pallas import tpu_sc as plsc

D_MODEL = 64
SCALE = math.sqrt(D_MODEL)

NC = 2
NS = 16
NW = NC * NS
LANES = 16

CHUNK = 512
KSTREAM = CHUNK // 128
NBUF = 2


@functools.partial(jax.jit, static_argnames=("n_rows",))
def _emb_lookup(idx2d, table, n_rows):
    rows_per_w = n_rows // NW
    n_chunks = rows_per_w // CHUNK

    mesh = plsc.VectorSubcoreMesh(
        core_axis_name="c", subcore_axis_name="s",
        num_cores=NC, num_subcores=NS)

    @functools.partial(
        pl.kernel,
        mesh=mesh,
        compiler_params=pltpu.CompilerParams(use_tc_tiling_on_sc=False),
        out_type=jax.ShapeDtypeStruct((n_rows, D_MODEL), jnp.float32),
        scratch_types=[
            pltpu.VMEM((NBUF, KSTREAM, 128), jnp.int32),
            pltpu.VMEM((NBUF, CHUNK, D_MODEL), jnp.float32),
            pltpu.SemaphoreType.DMA,
            pltpu.SemaphoreType.DMA,
            pltpu.SemaphoreType.DMA,
            pltpu.SemaphoreType.DMA,
        ],
    )
    def body(idx_hbm, table_hbm, out_hbm, idx_v, rows_v,
             gsem0, gsem1, ssem0, ssem1):
        gsems = (gsem0, gsem1)
        ssems = (ssem0, ssem1)
        wid = lax.axis_index("s") * NC + lax.axis_index("c")
        base = wid * rows_per_w
        idx_row0 = wid * (rows_per_w // 128)

        def start_gather(c, b):
            pltpu.sync_copy(
                idx_hbm.at[pl.ds(idx_row0 + c * KSTREAM, KSTREAM)],
                idx_v.at[b])
            for j in range(KSTREAM):
                pltpu.async_copy(
                    table_hbm.at[idx_v.at[b, j]],
                    rows_v.at[b, pl.ds(j * 128, 128)],
                    gsems[b])

        def wait_gather(b):
            pltpu.make_async_copy(
                table_hbm.at[pl.ds(0, CHUNK)], rows_v.at[b], gsems[b]).wait()

        def scale_chunk(b):
            def row_body(r, carry):
                for jj in range(D_MODEL // LANES):
                    sl = pl.ds(jj * LANES, LANES)
                    rows_v[b, r, sl] = rows_v[b, r, sl] * SCALE
                return carry
            lax.fori_loop(0, CHUNK, row_body, 0, unroll=4)

        def start_store(c, b):
            pltpu.async_copy(
                rows_v.at[b], out_hbm.at[pl.ds(base + c * CHUNK, CHUNK)],
                ssems[b])

        def wait_store(b):
            pltpu.make_async_copy(
                rows_v.at[b], out_hbm.at[pl.ds(0, CHUNK)], ssems[b]).wait()

        start_gather(0, 0)

        def group_body(i, carry):
            g = i * NBUF
            for b in range(NBUF):
                c = g + b
                nb = (b + 1) % NBUF
                nxt = c + 1

                @pl.when(nxt < n_chunks)
                def _prefetch():
                    @pl.when(nxt >= NBUF)
                    def _reclaim():
                        wait_store(nb)
                    start_gather(nxt, nb)

                wait_gather(b)
                scale_chunk(b)
                start_store(c, b)
            return carry

        lax.fori_loop(0, n_chunks // NBUF, group_body, 0)
        for b in range(NBUF):
            wait_store(b)

    return body(idx2d, table)


def kernel(X, table):
    n_rows = X.shape[0] * X.shape[1]
    idx2d = X.reshape(n_rows // 128, 128).astype(jnp.int32)
    out = _emb_lookup(idx2d, table, n_rows)
    return out.reshape(X.shape[0], X.shape[1], D_MODEL)


# speedup vs baseline: 1.0106x; 1.0106x over previous
import functools
import math

import jax
import jax.numpy as jnp
from jax import lax
from jax.experimental import pallas as pl
from jax.experimental.pallas import tpu as pltpu
from jax.experimental.pallas import tpu_sc as plsc

D_MODEL = 64
SCALE = math.sqrt(D_MODEL)

NC = 2
NS = 16
NW = NC * NS
LANES = 16

CHUNK_X = 4
NBUF = 2


@functools.partial(jax.jit, static_argnames=("seq",))
def _emb_lookup(idx, table, seq):
    n_x = idx.shape[0]
    x_per_w = n_x // NW
    n_chunks = x_per_w // CHUNK_X
    chunk_rows = CHUNK_X * seq
    rows_per_w = x_per_w * seq
    n_rows = n_x * seq
    splits = []
    off = 0
    while off < seq:
        w = min(128, seq - off)
        splits.append((off, w))
        off += w

    mesh = plsc.VectorSubcoreMesh(
        core_axis_name="c", subcore_axis_name="s",
        num_cores=NC, num_subcores=NS)

    @functools.partial(
        pl.kernel,
        mesh=mesh,
        compiler_params=pltpu.CompilerParams(use_tc_tiling_on_sc=False),
        out_type=jax.ShapeDtypeStruct((n_rows, D_MODEL), jnp.float32),
        scratch_types=[
            pltpu.VMEM((NBUF, CHUNK_X, seq), jnp.int32),
            pltpu.VMEM((NBUF, chunk_rows, D_MODEL), jnp.float32),
            pltpu.SemaphoreType.DMA,
            pltpu.SemaphoreType.DMA,
            pltpu.SemaphoreType.DMA,
            pltpu.SemaphoreType.DMA,
        ],
    )
    def body(idx_hbm, table_hbm, out_hbm, idx_v, rows_v,
             gsem0, gsem1, ssem0, ssem1):
        gsems = (gsem0, gsem1)
        ssems = (ssem0, ssem1)
        wid = lax.axis_index("s") * NC + lax.axis_index("c")
        xrow0 = wid * x_per_w
        base = wid * rows_per_w

        def start_gather(c, b):
            pltpu.sync_copy(
                idx_hbm.at[pl.ds(xrow0 + c * CHUNK_X, CHUNK_X)],
                idx_v.at[b])
            for r in range(CHUNK_X):
                for (o, w) in splits:
                    pltpu.async_copy(
                        table_hbm.at[idx_v.at[b, r, pl.ds(o, w)]],
                        rows_v.at[b, pl.ds(r * seq + o, w)],
                        gsems[b])

        def wait_gather(b):
            pltpu.make_async_copy(
                table_hbm.at[pl.ds(0, chunk_rows)], rows_v.at[b],
                gsems[b]).wait()

        def scale_chunk(b):
            def row_body(r, carry):
                for jj in range(D_MODEL // LANES):
                    sl = pl.ds(jj * LANES, LANES)
                    rows_v[b, r, sl] = rows_v[b, r, sl] * SCALE
                return carry
            lax.fori_loop(0, chunk_rows, row_body, 0, unroll=4)

        def start_store(c, b):
            pltpu.async_copy(
                rows_v.at[b],
                out_hbm.at[pl.ds(base + c * chunk_rows, chunk_rows)],
                ssems[b])

        def wait_store(b):
            pltpu.make_async_copy(
                rows_v.at[b], out_hbm.at[pl.ds(0, chunk_rows)],
                ssems[b]).wait()

        start_gather(0, 0)

        def group_body(i, carry):
            g = i * NBUF
            for b in range(NBUF):
                c = g + b
                nb = (b + 1) % NBUF
                nxt = c + 1

                @pl.when(nxt < n_chunks)
                def _prefetch():
                    @pl.when(nxt >= NBUF)
                    def _reclaim():
                        wait_store(nb)
                    start_gather(nxt, nb)

                wait_gather(b)
                scale_chunk(b)
                start_store(c, b)
            return carry

        lax.fori_loop(0, n_chunks // NBUF, group_body, 0)
        for b in range(NBUF):
            wait_store(b)

    return body(idx, table)


def kernel(X, table):
    idx = X.astype(jnp.int32)
    out = _emb_lookup(idx, table, X.shape[1])
    return out.reshape(X.shape[0], X.shape[1], D_MODEL)
